# baseline (speedup 1.0000x reference)
import functools

import jax
import jax.numpy as jnp
from jax import lax
from jax.experimental import pallas as pl
from jax.experimental.pallas import tpu as pltpu
from jax.experimental.pallas import tpu_sc as plsc

N_NODES = 10000
E_EDGES = 320000
D = 128
NC, NS = 2, 16
NW = NC * NS
K = 128
N_PAD = 10240
E_PAD = 327680
CHUNKS = E_PAD // (NW * K)
HALF = CHUNKS // 2
ROWS_PT = N_PAD // NS
ZCH = 32

_MESH = dict(core_axis_name="c", subcore_axis_name="s")


def _sc_degree(dst_idx, ones_k, zeros_deg):
    @functools.partial(
        pl.kernel,
        out_type=jax.ShapeDtypeStruct((NC, N_PAD), jnp.float32),
        mesh=plsc.VectorSubcoreMesh(**_MESH),
        scratch_types=[
            pltpu.VMEM((CHUNKS, K), jnp.int32),
            pltpu.VMEM((K,), jnp.float32),
            pltpu.VMEM_SHARED((N_PAD,), jnp.float32),
        ],
    )
    def deg_kernel(dst_hbm, ones_hbm, zdeg_hbm, out_hbm, idx_v, ones_v, deg_sh):
        c = lax.axis_index("c")
        s = lax.axis_index("s")
        wid = c * NS + s
        pltpu.sync_copy(zdeg_hbm, deg_sh.at[pl.ds(s * ROWS_PT, ROWS_PT)])
        pltpu.sync_copy(ones_hbm, ones_v)
        pltpu.sync_copy(dst_hbm.at[wid], idx_v)
        plsc.subcore_barrier()

        def body(j, carry):
            pltpu.sync_copy(ones_v, deg_sh.at[idx_v.at[j]], add=True)
            return carry

        lax.fori_loop(0, CHUNKS, body, 0)
        plsc.subcore_barrier()
        pltpu.sync_copy(
            deg_sh.at[pl.ds(s * ROWS_PT, ROWS_PT)],
            out_hbm.at[c, pl.ds(s * ROWS_PT, ROWS_PT)],
        )

    return deg_kernel(dst_idx, ones_k, zeros_deg)


def _sc_hop(g, src_idx, dst_idx, zeros_rows):
    @functools.partial(
        pl.kernel,
        out_type=jax.ShapeDtypeStruct((NC, N_PAD, D), jnp.float32),
        mesh=plsc.VectorSubcoreMesh(**_MESH),
        scratch_types=[
            pltpu.VMEM((HALF, K), jnp.int32),
            pltpu.VMEM((HALF, K), jnp.int32),
            pltpu.VMEM((K, D), jnp.float32),
            pltpu.VMEM((K, D), jnp.float32),
            pltpu.VMEM_SHARED((N_PAD, D), jnp.float32),
            pltpu.SemaphoreType.DMA,
            pltpu.SemaphoreType.DMA,
            pltpu.SemaphoreType.DMA,
        ],
    )
    def hop_kernel(g_hbm, src_hbm, dst_hbm, zrows_hbm, out_hbm,
                   si_v, di_v, buf0, buf1, agg_sh, sem0, sem1, zsem):
        c = lax.axis_index("c")
        s = lax.axis_index("s")
        wid = c * NS + s

        def zcp(i, carry):
            pltpu.async_copy(
                zrows_hbm, agg_sh.at[pl.ds(s * ROWS_PT + i * ZCH, ZCH)], zsem)
            return carry

        lax.fori_loop(0, ROWS_PT // ZCH, zcp, 0)

        def zdr(i, carry):
            pltpu.make_async_copy(
                zrows_hbm, agg_sh.at[pl.ds(s * ROWS_PT, ZCH)], zsem).wait()
            return carry

        lax.fori_loop(0, ROWS_PT // ZCH, zdr, 0)
        plsc.subcore_barrier()

        for h in range(2):
            pltpu.sync_copy(src_hbm.at[wid, pl.ds(h * HALF, HALF)], si_v)
            pltpu.sync_copy(dst_hbm.at[wid, pl.ds(h * HALF, HALF)], di_v)
            pltpu.async_copy(g_hbm.at[si_v.at[0]], buf0, sem0)

            def pair(i, carry):
                j = 2 * i
                pltpu.make_async_copy(g_hbm.at[si_v.at[j]], buf0, sem0).wait()
                pltpu.async_copy(g_hbm.at[si_v.at[j + 1]], buf1, sem1)
                pltpu.sync_copy(buf0, agg_sh.at[di_v.at[j]], add=True)
                pltpu.make_async_copy(g_hbm.at[si_v.at[j + 1]], buf1, sem1).wait()
                pltpu.async_copy(g_hbm.at[si_v.at[j + 2]], buf0, sem0)
                pltpu.sync_copy(buf1, agg_sh.at[di_v.at[j + 1]], add=True)
                return carry

            lax.fori_loop(0, (HALF - 2) // 2, pair, 0)
            pltpu.make_async_copy(g_hbm.at[si_v.at[HALF - 2]], buf0, sem0).wait()
            pltpu.async_copy(g_hbm.at[si_v.at[HALF - 1]], buf1, sem1)
            pltpu.sync_copy(buf0, agg_sh.at[di_v.at[HALF - 2]], add=True)
            pltpu.make_async_copy(g_hbm.at[si_v.at[HALF - 1]], buf1, sem1).wait()
            pltpu.sync_copy(buf1, agg_sh.at[di_v.at[HALF - 1]], add=True)
        plsc.subcore_barrier()
        pltpu.sync_copy(
            agg_sh.at[pl.ds(s * ROWS_PT, ROWS_PT)],
            out_hbm.at[c, pl.ds(s * ROWS_PT, ROWS_PT)],
        )

    return hop_kernel(g, src_idx, dst_idx, zeros_rows)


_R = 2000


def _tc_prep(deg_a, deg_b, feat):
    def body(da, db, f, norm_o, g0_o):
        deg = jnp.maximum(da[...] + db[...], 1.0)
        nrm = lax.rsqrt(deg)
        norm_o[...] = nrm
        g0_o[...] = f[...] * nrm

    return pl.pallas_call(
        body,
        grid=(N_NODES // _R,),
        in_specs=[
            pl.BlockSpec((_R, 1), lambda i: (i, 0)),
            pl.BlockSpec((_R, 1), lambda i: (i, 0)),
            pl.BlockSpec((_R, D), lambda i: (i, 0)),
        ],
        out_specs=[
            pl.BlockSpec((_R, 1), lambda i: (i, 0)),
            pl.BlockSpec((_R, D), lambda i: (i, 0)),
        ],
        out_shape=[
            jax.ShapeDtypeStruct((N_NODES, 1), jnp.float32),
            jax.ShapeDtypeStruct((N_PAD, D), jnp.float32),
        ],
    )(deg_a, deg_b, feat)


def _tc_mid(partials, norm):
    def body(p, nrm, h_o, g_o):
        h = (p[0] + p[1]) * nrm[...]
        h_o[...] = h
        g_o[...] = h * nrm[...]

    return pl.pallas_call(
        body,
        grid=(N_NODES // _R,),
        in_specs=[
            pl.BlockSpec((NC, _R, D), lambda i: (0, i, 0)),
            pl.BlockSpec((_R, 1), lambda i: (i, 0)),
        ],
        out_specs=[
            pl.BlockSpec((_R, D), lambda i: (i, 0)),
            pl.BlockSpec((_R, D), lambda i: (i, 0)),
        ],
        out_shape=[
            jax.ShapeDtypeStruct((N_NODES, D), jnp.float32),
            jax.ShapeDtypeStruct((N_PAD, D), jnp.float32),
        ],
    )(partials, norm)


def _tc_final(partials, norm, feat, h1, w, b2):
    def body(q, nrm, f, h, wr, br, o):
        h2 = (q[0] + q[1]) * nrm[...]
        wf = wr[...]
        acc = jnp.dot(f[...], wf[0:D], preferred_element_type=jnp.float32)
        acc = acc + jnp.dot(h[...], wf[D:2 * D], preferred_element_type=jnp.float32)
        acc = acc + jnp.dot(h2, wf[2 * D:3 * D], preferred_element_type=jnp.float32)
        o[...] = acc + br[...]

    return pl.pallas_call(
        body,
        grid=(N_NODES // _R,),
        in_specs=[
            pl.BlockSpec((NC, _R, D), lambda i: (0, i, 0)),
            pl.BlockSpec((_R, 1), lambda i: (i, 0)),
            pl.BlockSpec((_R, D), lambda i: (i, 0)),
            pl.BlockSpec((_R, D), lambda i: (i, 0)),
            pl.BlockSpec((3 * D, D), lambda i: (0, 0)),
            pl.BlockSpec((1, D), lambda i: (0, 0)),
        ],
        out_specs=pl.BlockSpec((_R, D), lambda i: (i, 0)),
        out_shape=jax.ShapeDtypeStruct((N_NODES, D), jnp.float32),
    )(partials, norm, feat, h1, w, b2)


def kernel(feat, edge_index, W, b):
    pad_ids = N_NODES + jnp.arange(E_PAD - E_EDGES, dtype=jnp.int32) % (
        N_PAD - N_NODES)
    pad_e = jnp.stack([pad_ids, pad_ids])
    ei = jnp.concatenate([edge_index, pad_e], axis=1)
    src = ei[0].reshape(NW, CHUNKS, K)
    dst = ei[1].reshape(NW, CHUNKS, K)
    ones_k = jnp.ones((K,), jnp.float32)
    zeros_deg = jnp.zeros((ROWS_PT,), jnp.float32)
    zeros_rows = jnp.zeros((ZCH, D), jnp.float32)

    deg_p = _sc_degree(dst, ones_k, zeros_deg)
    deg_a = deg_p[0].reshape(N_PAD, 1)[:N_NODES]
    deg_b = deg_p[1].reshape(N_PAD, 1)[:N_NODES]
    norm, g0 = _tc_prep(deg_a, deg_b, feat)
    p1 = _sc_hop(g0, src, dst, zeros_rows)
    h1, g1 = _tc_mid(p1, norm)
    p2 = _sc_hop(g1, src, dst, zeros_rows)
    return _tc_final(p2, norm, feat, h1, W, b.reshape(1, D))


# speedup vs baseline: 1.1350x; 1.1350x over previous
import functools

import jax
import jax.numpy as jnp
from jax import lax
from jax.experimental import pallas as pl
from jax.experimental.pallas import tpu as pltpu
from jax.experimental.pallas import tpu_sc as plsc

N_NODES = 10000
E_EDGES = 320000
D = 128
NC, NS = 2, 16
NW = NC * NS
K = 128
N_PAD = 10240
E_PAD = 327680
CHUNKS = E_PAD // (NW * K)
HALF = CHUNKS // 2
ROWS_PT = N_PAD // NS
ZCH = 32

_MESH = dict(core_axis_name="c", subcore_axis_name="s")


def _sc_degree(dst_idx, ones_k, zeros_deg):
    @functools.partial(
        pl.kernel,
        out_type=jax.ShapeDtypeStruct((NC, N_PAD), jnp.float32),
        mesh=plsc.VectorSubcoreMesh(**_MESH),
        scratch_types=[
            pltpu.VMEM((CHUNKS, K), jnp.int32),
            pltpu.VMEM((K,), jnp.float32),
            pltpu.VMEM_SHARED((N_PAD,), jnp.float32),
        ],
    )
    def deg_kernel(dst_hbm, ones_hbm, zdeg_hbm, out_hbm, idx_v, ones_v, deg_sh):
        c = lax.axis_index("c")
        s = lax.axis_index("s")
        wid = c * NS + s
        pltpu.sync_copy(zdeg_hbm, deg_sh.at[pl.ds(s * ROWS_PT, ROWS_PT)])
        pltpu.sync_copy(ones_hbm, ones_v)
        pltpu.sync_copy(dst_hbm.at[wid], idx_v)
        plsc.subcore_barrier()

        def body(j, carry):
            pltpu.sync_copy(ones_v, deg_sh.at[idx_v.at[j]], add=True)
            return carry

        lax.fori_loop(0, CHUNKS, body, 0)
        plsc.subcore_barrier()
        pltpu.sync_copy(
            deg_sh.at[pl.ds(s * ROWS_PT, ROWS_PT)],
            out_hbm.at[c, pl.ds(s * ROWS_PT, ROWS_PT)],
        )

    return deg_kernel(dst_idx, ones_k, zeros_deg)


def _sc_hop(g, src_idx, dst_idx, zeros_rows):
    @functools.partial(
        pl.kernel,
        out_type=jax.ShapeDtypeStruct((NC, N_PAD, D), jnp.float32),
        mesh=plsc.VectorSubcoreMesh(**_MESH),
        scratch_types=[
            pltpu.VMEM((HALF, K), jnp.int32),
            pltpu.VMEM((HALF, K), jnp.int32),
            pltpu.VMEM((K, D), jnp.float32),
            pltpu.VMEM((K, D), jnp.float32),
            pltpu.VMEM_SHARED((N_PAD, D), jnp.float32),
            pltpu.SemaphoreType.DMA,
            pltpu.SemaphoreType.DMA,
            pltpu.SemaphoreType.DMA,
        ],
    )
    def hop_kernel(g_hbm, src_hbm, dst_hbm, zrows_hbm, out_hbm,
                   si_v, di_v, buf0, buf1, agg_sh, sem0, sem1, zsem):
        c = lax.axis_index("c")
        s = lax.axis_index("s")
        wid = c * NS + s

        def zcp(i, carry):
            pltpu.async_copy(
                zrows_hbm, agg_sh.at[pl.ds(s * ROWS_PT + i * ZCH, ZCH)], zsem)
            return carry

        lax.fori_loop(0, ROWS_PT // ZCH, zcp, 0)

        def zdr(i, carry):
            pltpu.make_async_copy(
                zrows_hbm, agg_sh.at[pl.ds(s * ROWS_PT, ZCH)], zsem).wait()
            return carry

        lax.fori_loop(0, ROWS_PT // ZCH, zdr, 0)
        plsc.subcore_barrier()

        for h in range(2):
            pltpu.sync_copy(src_hbm.at[wid, pl.ds(h * HALF, HALF)], si_v)
            pltpu.sync_copy(dst_hbm.at[wid, pl.ds(h * HALF, HALF)], di_v)
            pltpu.async_copy(g_hbm.at[si_v.at[0]], buf0, sem0)
            pltpu.async_copy(g_hbm.at[si_v.at[1]], buf1, sem1)

            def pair(i, carry):
                j = 2 * i
                pltpu.make_async_copy(g_hbm.at[si_v.at[j]], buf0, sem0).wait()
                pltpu.sync_copy(buf0, agg_sh.at[di_v.at[j]], add=True)
                pltpu.async_copy(g_hbm.at[si_v.at[j + 2]], buf0, sem0)
                pltpu.make_async_copy(g_hbm.at[si_v.at[j + 1]], buf1, sem1).wait()
                pltpu.sync_copy(buf1, agg_sh.at[di_v.at[j + 1]], add=True)
                pltpu.async_copy(g_hbm.at[si_v.at[j + 3]], buf1, sem1)
                return carry

            lax.fori_loop(0, (HALF - 2) // 2, pair, 0)
            pltpu.make_async_copy(g_hbm.at[si_v.at[HALF - 2]], buf0, sem0).wait()
            pltpu.sync_copy(buf0, agg_sh.at[di_v.at[HALF - 2]], add=True)
            pltpu.make_async_copy(g_hbm.at[si_v.at[HALF - 1]], buf1, sem1).wait()
            pltpu.sync_copy(buf1, agg_sh.at[di_v.at[HALF - 1]], add=True)
        plsc.subcore_barrier()
        pltpu.sync_copy(
            agg_sh.at[pl.ds(s * ROWS_PT, ROWS_PT)],
            out_hbm.at[c, pl.ds(s * ROWS_PT, ROWS_PT)],
        )

    return hop_kernel(g, src_idx, dst_idx, zeros_rows)


_R = 2048


def _tc_prep(deg_a, deg_b, feat):
    def body(da, db, f, norm_o, g0_o):
        deg = jnp.maximum(da[...] + db[...], 1.0)
        nrm = lax.rsqrt(deg)
        norm_o[...] = nrm
        g0_o[...] = f[...] * nrm

    return pl.pallas_call(
        body,
        grid=(N_PAD // _R,),
        in_specs=[
            pl.BlockSpec((_R, 1), lambda i: (i, 0)),
            pl.BlockSpec((_R, 1), lambda i: (i, 0)),
            pl.BlockSpec((_R, D), lambda i: (i, 0)),
        ],
        out_specs=[
            pl.BlockSpec((_R, 1), lambda i: (i, 0)),
            pl.BlockSpec((_R, D), lambda i: (i, 0)),
        ],
        out_shape=[
            jax.ShapeDtypeStruct((N_PAD, 1), jnp.float32),
            jax.ShapeDtypeStruct((N_PAD, D), jnp.float32),
        ],
    )(deg_a, deg_b, feat)


def _tc_mid(partials, norm):
    def body(p, nrm, h_o, g_o):
        h = (p[0] + p[1]) * nrm[...]
        h_o[...] = h
        g_o[...] = h * nrm[...]

    return pl.pallas_call(
        body,
        grid=(N_PAD // _R,),
        in_specs=[
            pl.BlockSpec((NC, _R, D), lambda i: (0, i, 0)),
            pl.BlockSpec((_R, 1), lambda i: (i, 0)),
        ],
        out_specs=[
            pl.BlockSpec((_R, D), lambda i: (i, 0)),
            pl.BlockSpec((_R, D), lambda i: (i, 0)),
        ],
        out_shape=[
            jax.ShapeDtypeStruct((N_PAD, D), jnp.float32),
            jax.ShapeDtypeStruct((N_PAD, D), jnp.float32),
        ],
    )(partials, norm)


def _tc_final(partials, norm, feat, h1, w, b2):
    def body(q, nrm, f, h, wr, br, o):
        h2 = (q[0] + q[1]) * nrm[...]
        wf = wr[...]
        acc = jnp.dot(f[...], wf[0:D], preferred_element_type=jnp.float32)
        acc = acc + jnp.dot(h[...], wf[D:2 * D], preferred_element_type=jnp.float32)
        acc = acc + jnp.dot(h2, wf[2 * D:3 * D], preferred_element_type=jnp.float32)
        o[...] = acc + br[...]

    return pl.pallas_call(
        body,
        grid=(N_PAD // _R,),
        in_specs=[
            pl.BlockSpec((NC, _R, D), lambda i: (0, i, 0)),
            pl.BlockSpec((_R, 1), lambda i: (i, 0)),
            pl.BlockSpec((_R, D), lambda i: (i, 0)),
            pl.BlockSpec((_R, D), lambda i: (i, 0)),
            pl.BlockSpec((3 * D, D), lambda i: (0, 0)),
            pl.BlockSpec((1, D), lambda i: (0, 0)),
        ],
        out_specs=pl.BlockSpec((_R, D), lambda i: (i, 0)),
        out_shape=jax.ShapeDtypeStruct((N_PAD, D), jnp.float32),
    )(partials, norm, feat, h1, w, b2)


def kernel(feat, edge_index, W, b):
    pad_ids = N_NODES + jnp.arange(E_PAD - E_EDGES, dtype=jnp.int32) % (
        N_PAD - N_NODES)
    pad_e = jnp.stack([pad_ids, pad_ids])
    ei = jnp.concatenate([edge_index, pad_e], axis=1)
    src = ei[0].reshape(NW, CHUNKS, K)
    dst = ei[1].reshape(NW, CHUNKS, K)
    ones_k = jnp.ones((K,), jnp.float32)
    zeros_deg = jnp.zeros((ROWS_PT,), jnp.float32)
    zeros_rows = jnp.zeros((ZCH, D), jnp.float32)
    feat_p = jnp.pad(feat, ((0, N_PAD - N_NODES), (0, 0)))

    deg_p = _sc_degree(dst, ones_k, zeros_deg)
    deg_a = deg_p[0].reshape(N_PAD, 1)
    deg_b = deg_p[1].reshape(N_PAD, 1)
    norm, g0 = _tc_prep(deg_a, deg_b, feat_p)
    p1 = _sc_hop(g0, src, dst, zeros_rows)
    h1, g1 = _tc_mid(p1, norm)
    p2 = _sc_hop(g1, src, dst, zeros_rows)
    out = _tc_final(p2, norm, feat_p, h1, W, b.reshape(1, D))
    return out[:N_NODES]


# speedup vs baseline: 1.1494x; 1.0127x over previous
import functools

import jax
import jax.numpy as jnp
from jax import lax
from jax.experimental import pallas as pl
from jax.experimental.pallas import tpu as pltpu
from jax.experimental.pallas import tpu_sc as plsc

N_NODES = 10000
E_EDGES = 320000
D = 128
NC, NS = 2, 16
NW = NC * NS
K = 128
N_PAD = 10240
E_PAD = 327680
CHUNKS = E_PAD // (NW * K)
HALF = CHUNKS // 2
ROWS_PT = N_PAD // NS
ZCH = 32

_MESH = dict(core_axis_name="c", subcore_axis_name="s")


def _sc_degree(dst_idx, ones_k, zeros_deg):
    @functools.partial(
        pl.kernel,
        out_type=jax.ShapeDtypeStruct((NC, N_PAD), jnp.float32),
        mesh=plsc.VectorSubcoreMesh(**_MESH),
        scratch_types=[
            pltpu.VMEM((CHUNKS, K), jnp.int32),
            pltpu.VMEM((K,), jnp.float32),
            pltpu.VMEM_SHARED((N_PAD,), jnp.float32),
            pltpu.SemaphoreType.DMA,
        ],
    )
    def deg_kernel(dst_hbm, ones_hbm, zdeg_hbm, out_hbm, idx_v, ones_v, deg_sh,
                   sem):
        c = lax.axis_index("c")
        s = lax.axis_index("s")
        wid = c * NS + s
        pltpu.sync_copy(zdeg_hbm, deg_sh.at[pl.ds(s * ROWS_PT, ROWS_PT)])
        pltpu.sync_copy(ones_hbm, ones_v)
        pltpu.sync_copy(dst_hbm.at[wid], idx_v)
        plsc.subcore_barrier()

        def body(j, carry):
            pltpu.async_copy(ones_v, deg_sh.at[idx_v.at[j]], add=True, sem=sem)
            return carry

        lax.fori_loop(0, CHUNKS, body, 0)

        def drain(j, carry):
            pltpu.make_async_copy(ones_v, deg_sh.at[idx_v.at[0]], sem).wait()
            return carry

        lax.fori_loop(0, CHUNKS, drain, 0)
        plsc.subcore_barrier()
        pltpu.sync_copy(
            deg_sh.at[pl.ds(s * ROWS_PT, ROWS_PT)],
            out_hbm.at[c, pl.ds(s * ROWS_PT, ROWS_PT)],
        )

    return deg_kernel(dst_idx, ones_k, zeros_deg)


def _sc_hop(g, src_idx, dst_idx, zeros_rows):
    @functools.partial(
        pl.kernel,
        out_type=jax.ShapeDtypeStruct((NC, N_PAD, D), jnp.float32),
        mesh=plsc.VectorSubcoreMesh(**_MESH),
        scratch_types=[
            pltpu.VMEM((HALF, K), jnp.int32),
            pltpu.VMEM((HALF, K), jnp.int32),
            pltpu.VMEM((K, D), jnp.float32),
            pltpu.VMEM((K, D), jnp.float32),
            pltpu.VMEM_SHARED((N_PAD, D), jnp.float32),
            pltpu.SemaphoreType.DMA,
            pltpu.SemaphoreType.DMA,
            pltpu.SemaphoreType.DMA,
        ],
    )
    def hop_kernel(g_hbm, src_hbm, dst_hbm, zrows_hbm, out_hbm,
                   si_v, di_v, buf0, buf1, agg_sh, sem0, sem1, zsem):
        c = lax.axis_index("c")
        s = lax.axis_index("s")
        wid = c * NS + s

        def zcp(i, carry):
            pltpu.async_copy(
                zrows_hbm, agg_sh.at[pl.ds(s * ROWS_PT + i * ZCH, ZCH)], zsem)
            return carry

        lax.fori_loop(0, ROWS_PT // ZCH, zcp, 0)

        def zdr(i, carry):
            pltpu.make_async_copy(
                zrows_hbm, agg_sh.at[pl.ds(s * ROWS_PT, ZCH)], zsem).wait()
            return carry

        lax.fori_loop(0, ROWS_PT // ZCH, zdr, 0)
        plsc.subcore_barrier()

        for h in range(2):
            pltpu.sync_copy(src_hbm.at[wid, pl.ds(h * HALF, HALF)], si_v)
            pltpu.sync_copy(dst_hbm.at[wid, pl.ds(h * HALF, HALF)], di_v)
            pltpu.async_copy(g_hbm.at[si_v.at[0]], buf0, sem0)
            pltpu.async_copy(g_hbm.at[si_v.at[1]], buf1, sem1)

            def pair(i, carry):
                j = 2 * i
                pltpu.make_async_copy(g_hbm.at[si_v.at[j]], buf0, sem0).wait()
                pltpu.sync_copy(buf0, agg_sh.at[di_v.at[j]], add=True)
                pltpu.async_copy(g_hbm.at[si_v.at[j + 2]], buf0, sem0)
                pltpu.make_async_copy(g_hbm.at[si_v.at[j + 1]], buf1, sem1).wait()
                pltpu.sync_copy(buf1, agg_sh.at[di_v.at[j + 1]], add=True)
                pltpu.async_copy(g_hbm.at[si_v.at[j + 3]], buf1, sem1)
                return carry

            lax.fori_loop(0, (HALF - 2) // 2, pair, 0)
            pltpu.make_async_copy(g_hbm.at[si_v.at[HALF - 2]], buf0, sem0).wait()
            pltpu.sync_copy(buf0, agg_sh.at[di_v.at[HALF - 2]], add=True)
            pltpu.make_async_copy(g_hbm.at[si_v.at[HALF - 1]], buf1, sem1).wait()
            pltpu.sync_copy(buf1, agg_sh.at[di_v.at[HALF - 1]], add=True)
        plsc.subcore_barrier()
        pltpu.sync_copy(
            agg_sh.at[pl.ds(s * ROWS_PT, ROWS_PT)],
            out_hbm.at[c, pl.ds(s * ROWS_PT, ROWS_PT)],
        )

    return hop_kernel(g, src_idx, dst_idx, zeros_rows)


_R = 2048


def _tc_prep(deg_a, deg_b, feat):
    def body(da, db, f, norm_o, g0_o):
        deg = jnp.maximum(da[...] + db[...], 1.0)
        nrm = lax.rsqrt(deg)
        norm_o[...] = nrm
        g0_o[...] = f[...] * nrm

    return pl.pallas_call(
        body,
        grid=(N_PAD // _R,),
        in_specs=[
            pl.BlockSpec((_R, 1), lambda i: (i, 0)),
            pl.BlockSpec((_R, 1), lambda i: (i, 0)),
            pl.BlockSpec((_R, D), lambda i: (i, 0)),
        ],
        out_specs=[
            pl.BlockSpec((_R, 1), lambda i: (i, 0)),
            pl.BlockSpec((_R, D), lambda i: (i, 0)),
        ],
        out_shape=[
            jax.ShapeDtypeStruct((N_PAD, 1), jnp.float32),
            jax.ShapeDtypeStruct((N_PAD, D), jnp.float32),
        ],
    )(deg_a, deg_b, feat)


def _tc_mid(partials, norm):
    def body(p, nrm, h_o, g_o):
        h = (p[0] + p[1]) * nrm[...]
        h_o[...] = h
        g_o[...] = h * nrm[...]

    return pl.pallas_call(
        body,
        grid=(N_PAD // _R,),
        in_specs=[
            pl.BlockSpec((NC, _R, D), lambda i: (0, i, 0)),
            pl.BlockSpec((_R, 1), lambda i: (i, 0)),
        ],
        out_specs=[
            pl.BlockSpec((_R, D), lambda i: (i, 0)),
            pl.BlockSpec((_R, D), lambda i: (i, 0)),
        ],
        out_shape=[
            jax.ShapeDtypeStruct((N_PAD, D), jnp.float32),
            jax.ShapeDtypeStruct((N_PAD, D), jnp.float32),
        ],
    )(partials, norm)


def _tc_final(partials, norm, feat, h1, w, b2):
    def body(q, nrm, f, h, wr, br, o):
        h2 = (q[0] + q[1]) * nrm[...]
        wf = wr[...]
        acc = jnp.dot(f[...], wf[0:D], preferred_element_type=jnp.float32)
        acc = acc + jnp.dot(h[...], wf[D:2 * D], preferred_element_type=jnp.float32)
        acc = acc + jnp.dot(h2, wf[2 * D:3 * D], preferred_element_type=jnp.float32)
        o[...] = acc + br[...]

    return pl.pallas_call(
        body,
        grid=(N_PAD // _R,),
        in_specs=[
            pl.BlockSpec((NC, _R, D), lambda i: (0, i, 0)),
            pl.BlockSpec((_R, 1), lambda i: (i, 0)),
            pl.BlockSpec((_R, D), lambda i: (i, 0)),
            pl.BlockSpec((_R, D), lambda i: (i, 0)),
            pl.BlockSpec((3 * D, D), lambda i: (0, 0)),
            pl.BlockSpec((1, D), lambda i: (0, 0)),
        ],
        out_specs=pl.BlockSpec((_R, D), lambda i: (i, 0)),
        out_shape=jax.ShapeDtypeStruct((N_PAD, D), jnp.float32),
    )(partials, norm, feat, h1, w, b2)


def kernel(feat, edge_index, W, b):
    pad_ids = N_NODES + jnp.arange(E_PAD - E_EDGES, dtype=jnp.int32) % (
        N_PAD - N_NODES)
    pad_e = jnp.stack([pad_ids, pad_ids])
    ei = jnp.concatenate([edge_index, pad_e], axis=1)
    src = ei[0].reshape(NW, CHUNKS, K)
    dst = ei[1].reshape(NW, CHUNKS, K)
    ones_k = jnp.ones((K,), jnp.float32)
    zeros_deg = jnp.zeros((ROWS_PT,), jnp.float32)
    zeros_rows = jnp.zeros((ZCH, D), jnp.float32)
    feat_p = jnp.pad(feat, ((0, N_PAD - N_NODES), (0, 0)))

    deg_p = _sc_degree(dst, ones_k, zeros_deg)
    deg_a = deg_p[0].reshape(N_PAD, 1)
    deg_b = deg_p[1].reshape(N_PAD, 1)
    norm, g0 = _tc_prep(deg_a, deg_b, feat_p)
    p1 = _sc_hop(g0, src, dst, zeros_rows)
    h1, g1 = _tc_mid(p1, norm)
    p2 = _sc_hop(g1, src, dst, zeros_rows)
    out = _tc_final(p2, norm, feat_p, h1, W, b.reshape(1, D))
    return out[:N_NODES]


# speedup vs baseline: 1.1588x; 1.0082x over previous
import functools

import jax
import jax.numpy as jnp
from jax import lax
from jax.experimental import pallas as pl
from jax.experimental.pallas import tpu as pltpu
from jax.experimental.pallas import tpu_sc as plsc

N_NODES = 10000
E_EDGES = 320000
D = 128
NC, NS = 2, 16
NW = NC * NS
K = 128
N_PAD = 10240
E_PAD = 327680
CHUNKS = E_PAD // (NW * K)
HALF = CHUNKS // 2
ROWS_PT = N_PAD // NS
ZCH = 32

_MESH = dict(core_axis_name="c", subcore_axis_name="s")


def _sc_degree(dst_idx, ones_k, zeros_deg):
    @functools.partial(
        pl.kernel,
        out_type=jax.ShapeDtypeStruct((NC, N_PAD), jnp.float32),
        mesh=plsc.VectorSubcoreMesh(**_MESH),
        scratch_types=[
            pltpu.VMEM((CHUNKS, K), jnp.int32),
            pltpu.VMEM((K,), jnp.float32),
            pltpu.VMEM_SHARED((N_PAD,), jnp.float32),
            pltpu.SemaphoreType.DMA,
        ],
    )
    def deg_kernel(dst_hbm, ones_hbm, zdeg_hbm, out_hbm, idx_v, ones_v, deg_sh,
                   sem):
        c = lax.axis_index("c")
        s = lax.axis_index("s")
        wid = c * NS + s
        pltpu.sync_copy(zdeg_hbm, deg_sh.at[pl.ds(s * ROWS_PT, ROWS_PT)])
        pltpu.sync_copy(ones_hbm, ones_v)
        pltpu.sync_copy(dst_hbm.at[wid], idx_v)
        plsc.subcore_barrier()

        def body(j, carry):
            pltpu.async_copy(ones_v, deg_sh.at[idx_v.at[j]], add=True, sem=sem)
            return carry

        lax.fori_loop(0, CHUNKS, body, 0)

        def drain(j, carry):
            pltpu.make_async_copy(ones_v, deg_sh.at[idx_v.at[0]], sem).wait()
            return carry

        lax.fori_loop(0, CHUNKS, drain, 0)
        plsc.subcore_barrier()
        pltpu.sync_copy(
            deg_sh.at[pl.ds(s * ROWS_PT, ROWS_PT)],
            out_hbm.at[c, pl.ds(s * ROWS_PT, ROWS_PT)],
        )

    return deg_kernel(dst_idx, ones_k, zeros_deg)


def _sc_hop(g, src_idx, dst_idx, zeros_rows):
    @functools.partial(
        pl.kernel,
        out_type=jax.ShapeDtypeStruct((NC, N_PAD, D), jnp.float32),
        mesh=plsc.VectorSubcoreMesh(**_MESH),
        scratch_types=[
            pltpu.VMEM((HALF, K), jnp.int32),
            pltpu.VMEM((HALF, K), jnp.int32),
            pltpu.VMEM((K, D), jnp.float32),
            pltpu.VMEM((K, D), jnp.float32),
            pltpu.VMEM_SHARED((N_PAD, D), jnp.float32),
            pltpu.SemaphoreType.DMA,
            pltpu.SemaphoreType.DMA,
            pltpu.SemaphoreType.DMA,
        ],
    )
    def hop_kernel(g_hbm, src_hbm, dst_hbm, zrows_hbm, out_hbm,
                   si_v, di_v, buf0, buf1, agg_sh, sem0, sem1, zsem):
        c = lax.axis_index("c")
        s = lax.axis_index("s")
        wid = c * NS + s

        def zcp(i, carry):
            pltpu.async_copy(
                zrows_hbm, agg_sh.at[pl.ds(s * ROWS_PT + i * ZCH, ZCH)], zsem)
            return carry

        lax.fori_loop(0, ROWS_PT // ZCH, zcp, 0)

        for h in range(2):
            pltpu.sync_copy(src_hbm.at[wid, pl.ds(h * HALF, HALF)], si_v)
            pltpu.sync_copy(dst_hbm.at[wid, pl.ds(h * HALF, HALF)], di_v)
            pltpu.async_copy(g_hbm.at[si_v.at[0]], buf0, sem0)
            pltpu.async_copy(g_hbm.at[si_v.at[1]], buf1, sem1)

            if h == 0:
                def zdr(i, carry):
                    pltpu.make_async_copy(
                        zrows_hbm, agg_sh.at[pl.ds(s * ROWS_PT, ZCH)],
                        zsem).wait()
                    return carry

                lax.fori_loop(0, ROWS_PT // ZCH, zdr, 0)
                plsc.subcore_barrier()

            def pair(i, carry):
                j = 2 * i
                pltpu.make_async_copy(g_hbm.at[si_v.at[j]], buf0, sem0).wait()
                pltpu.sync_copy(buf0, agg_sh.at[di_v.at[j]], add=True)
                pltpu.async_copy(g_hbm.at[si_v.at[j + 2]], buf0, sem0)
                pltpu.make_async_copy(g_hbm.at[si_v.at[j + 1]], buf1, sem1).wait()
                pltpu.sync_copy(buf1, agg_sh.at[di_v.at[j + 1]], add=True)
                pltpu.async_copy(g_hbm.at[si_v.at[j + 3]], buf1, sem1)
                return carry

            lax.fori_loop(0, (HALF - 2) // 2, pair, 0)
            pltpu.make_async_copy(g_hbm.at[si_v.at[HALF - 2]], buf0, sem0).wait()
            pltpu.sync_copy(buf0, agg_sh.at[di_v.at[HALF - 2]], add=True)
            pltpu.make_async_copy(g_hbm.at[si_v.at[HALF - 1]], buf1, sem1).wait()
            pltpu.sync_copy(buf1, agg_sh.at[di_v.at[HALF - 1]], add=True)
        plsc.subcore_barrier()
        pltpu.sync_copy(
            agg_sh.at[pl.ds(s * ROWS_PT, ROWS_PT)],
            out_hbm.at[c, pl.ds(s * ROWS_PT, ROWS_PT)],
        )

    return hop_kernel(g, src_idx, dst_idx, zeros_rows)


_R = 2048


def _tc_prep(deg_a, deg_b, feat):
    def body(da, db, f, norm_o, g0_o):
        deg = jnp.maximum(da[...] + db[...], 1.0)
        nrm = lax.rsqrt(deg)
        norm_o[...] = nrm
        g0_o[...] = f[...] * nrm

    return pl.pallas_call(
        body,
        grid=(N_PAD // _R,),
        in_specs=[
            pl.BlockSpec((_R, 1), lambda i: (i, 0)),
            pl.BlockSpec((_R, 1), lambda i: (i, 0)),
            pl.BlockSpec((_R, D), lambda i: (i, 0)),
        ],
        out_specs=[
            pl.BlockSpec((_R, 1), lambda i: (i, 0)),
            pl.BlockSpec((_R, D), lambda i: (i, 0)),
        ],
        out_shape=[
            jax.ShapeDtypeStruct((N_PAD, 1), jnp.float32),
            jax.ShapeDtypeStruct((N_PAD, D), jnp.float32),
        ],
    )(deg_a, deg_b, feat)


def _tc_mid(partials, norm):
    def body(p, nrm, h_o, g_o):
        h = (p[0] + p[1]) * nrm[...]
        h_o[...] = h
        g_o[...] = h * nrm[...]

    return pl.pallas_call(
        body,
        grid=(N_PAD // _R,),
        in_specs=[
            pl.BlockSpec((NC, _R, D), lambda i: (0, i, 0)),
            pl.BlockSpec((_R, 1), lambda i: (i, 0)),
        ],
        out_specs=[
            pl.BlockSpec((_R, D), lambda i: (i, 0)),
            pl.BlockSpec((_R, D), lambda i: (i, 0)),
        ],
        out_shape=[
            jax.ShapeDtypeStruct((N_PAD, D), jnp.float32),
            jax.ShapeDtypeStruct((N_PAD, D), jnp.float32),
        ],
    )(partials, norm)


def _tc_final(partials, norm, feat, h1, w, b2):
    def body(q, nrm, f, h, wr, br, o):
        h2 = (q[0] + q[1]) * nrm[...]
        wf = wr[...]
        acc = jnp.dot(f[...], wf[0:D], preferred_element_type=jnp.float32)
        acc = acc + jnp.dot(h[...], wf[D:2 * D], preferred_element_type=jnp.float32)
        acc = acc + jnp.dot(h2, wf[2 * D:3 * D], preferred_element_type=jnp.float32)
        o[...] = acc + br[...]

    return pl.pallas_call(
        body,
        grid=(N_PAD // _R,),
        in_specs=[
            pl.BlockSpec((NC, _R, D), lambda i: (0, i, 0)),
            pl.BlockSpec((_R, 1), lambda i: (i, 0)),
            pl.BlockSpec((_R, D), lambda i: (i, 0)),
            pl.BlockSpec((_R, D), lambda i: (i, 0)),
            pl.BlockSpec((3 * D, D), lambda i: (0, 0)),
            pl.BlockSpec((1, D), lambda i: (0, 0)),
        ],
        out_specs=pl.BlockSpec((_R, D), lambda i: (i, 0)),
        out_shape=jax.ShapeDtypeStruct((N_PAD, D), jnp.float32),
    )(partials, norm, feat, h1, w, b2)


def kernel(feat, edge_index, W, b):
    pad_ids = N_NODES + jnp.arange(E_PAD - E_EDGES, dtype=jnp.int32) % (
        N_PAD - N_NODES)
    pad_e = jnp.stack([pad_ids, pad_ids])
    ei = jnp.concatenate([edge_index, pad_e], axis=1)
    src = ei[0].reshape(NW, CHUNKS, K)
    dst = ei[1].reshape(NW, CHUNKS, K)
    ones_k = jnp.ones((K,), jnp.float32)
    zeros_deg = jnp.zeros((ROWS_PT,), jnp.float32)
    zeros_rows = jnp.zeros((ZCH, D), jnp.float32)
    feat_p = jnp.pad(feat, ((0, N_PAD - N_NODES), (0, 0)))

    deg_p = _sc_degree(dst, ones_k, zeros_deg)
    deg_a = deg_p[0].reshape(N_PAD, 1)
    deg_b = deg_p[1].reshape(N_PAD, 1)
    norm, g0 = _tc_prep(deg_a, deg_b, feat_p)
    p1 = _sc_hop(g0, src, dst, zeros_rows)
    h1, g1 = _tc_mid(p1, norm)
    p2 = _sc_hop(g1, src, dst, zeros_rows)
    out = _tc_final(p2, norm, feat_p, h1, W, b.reshape(1, D))
    return out[:N_NODES]
